# baseline (speedup 1.0000x reference)
import functools

import jax
import jax.numpy as jnp
from jax import lax
from jax.experimental import pallas as pl
from jax.experimental.pallas import tpu as pltpu
from jax.experimental.pallas import tpu_sc as plsc

_N = 10000
_E = 160000
_NI = 10
_HID = 256
_L = 16
_NT = 16
_NP = 10240
_SL = _NP // _NT
_SLV = _SL // _L
_EP = _E // _NT
_EPV = _EP // _L


def _v0_body(h_ref, we_ref, be_ref, wl_ref, bl_ref, wh_ref, out_ref):
    u = jnp.dot(wl_ref[...], wh_ref[...], preferred_element_type=jnp.float32)
    wv = jnp.dot(we_ref[...], u, preferred_element_type=jnp.float32)
    cv = (jnp.dot(be_ref[...], u, preferred_element_type=jnp.float32)
          + jnp.dot(bl_ref[...], wh_ref[...], preferred_element_type=jnp.float32))
    out_ref[...] = jnp.dot(h_ref[...], wv, preferred_element_type=jnp.float32) + cv


def _compute_v0(h, W_emb, b_emb, W_l0, b_l0, W_halt):
    nb = 10
    rb = _N // nb
    full = lambda shape: pl.BlockSpec(shape, lambda i: (0, 0))
    return pl.pallas_call(
        _v0_body,
        grid=(nb,),
        in_specs=[
            pl.BlockSpec((rb, _HID), lambda i: (i, 0)),
            full((_HID, _HID)),
            full((1, _HID)),
            full((_HID, _HID)),
            full((1, _HID)),
            full((_HID, 1)),
        ],
        out_specs=pl.BlockSpec((rb, 1), lambda i: (i, 0)),
        out_shape=jax.ShapeDtypeStruct((_N, 1), jnp.float32),
    )(h, W_emb, b_emb[None, :], W_l0, b_l0[None, :], W_halt)


def _readout_body(s_ref, h_ref, we_ref, be_ref, wl_ref, bl_ref,
                  wr0_ref, br0_ref, wr1_ref, br1_ref, wr2_ref, br2_ref,
                  out_ref, t_acc, ss_acc):
    g = pl.program_id(0)

    @pl.when(g == 0)
    def _():
        t_acc[...] = jnp.zeros_like(t_acc)
        ss_acc[...] = jnp.zeros_like(ss_acc)

    t_acc[...] += lax.dot_general(
        s_ref[...], h_ref[...], (((0,), (0,)), ((), ())),
        preferred_element_type=jnp.float32)
    ss_acc[...] += jnp.sum(s_ref[...]).reshape(1, 1)

    @pl.when(g == pl.num_programs(0) - 1)
    def _():
        t = t_acc[...]
        ss = ss_acc[...]
        tw = jnp.dot(jnp.dot(t, we_ref[...], preferred_element_type=jnp.float32),
                     wl_ref[...], preferred_element_type=jnp.float32)
        bb = (jnp.dot(be_ref[...], wl_ref[...], preferred_element_type=jnp.float32)
              + bl_ref[...])
        hg = (tw + ss * bb) * (1.0 / _N)
        y = jnp.maximum(
            jnp.dot(hg, wr0_ref[...], preferred_element_type=jnp.float32)
            + br0_ref[...], 0.0)
        y = jnp.maximum(
            jnp.dot(y, wr1_ref[...], preferred_element_type=jnp.float32)
            + br1_ref[...], 0.0)
        out_ref[...] = (jnp.dot(y, wr2_ref[...], preferred_element_type=jnp.float32)
                        + br2_ref[...])


def _readout(s, h, W_emb, b_emb, W_l0, b_l0, Wr0, br0, Wr1, br1, Wr2, br2):
    nb = 10
    rb = _N // nb
    full = lambda shape: pl.BlockSpec(shape, lambda i: (0, 0))
    return pl.pallas_call(
        _readout_body,
        grid=(nb,),
        in_specs=[
            pl.BlockSpec((rb, 1), lambda i: (i, 0)),
            pl.BlockSpec((rb, _HID), lambda i: (i, 0)),
            full((_HID, _HID)),
            full((1, _HID)),
            full((_HID, _HID)),
            full((1, _HID)),
            full((_HID, _HID // 2)),
            full((1, _HID // 2)),
            full((_HID // 2, _HID // 4)),
            full((1, _HID // 4)),
            full((_HID // 4, 10)),
            full((1, 10)),
        ],
        out_specs=full((1, 10)),
        out_shape=jax.ShapeDtypeStruct((1, 10), jnp.float32),
        scratch_shapes=[
            pltpu.VMEM((1, _HID), jnp.float32),
            pltpu.VMEM((1, 1), jnp.float32),
        ],
    )(s, h, W_emb, b_emb[None, :], W_l0, b_l0[None, :],
      Wr0, br0[None, :], Wr1, br1[None, :], Wr2, br2[None, :])


def _sc_body(src_h, dst_h, v0_h, b_h,
             steps_h, rem_h, s_h,
             srcv, dstv, pkv, vbuf, pbuf, idv, accv, zslice,
             disv, sumh, stepsv, contv, kv, pv, vslice, remv, cbuf, bv,
             sp_acc, sp_v, dmasem):
    tid = lax.axis_index("s")
    e0 = tid * _EP
    n0 = tid * _SL

    pltpu.sync_copy(src_h.at[pl.ds(e0, _EP)], srcv)
    pltpu.sync_copy(dst_h.at[pl.ds(e0, _EP)], dstv)
    pltpu.sync_copy(b_h, bv)

    zero16 = jnp.zeros((_L,), jnp.float32)
    one16 = jnp.ones((_L,), jnp.float32)

    @plsc.parallel_loop(0, _NP // _L, unroll=8)
    def idv_b(j):
        idv[pl.ds(j * _L, _L)] = j * _L + lax.iota(jnp.int32, _L)

    @plsc.parallel_loop(0, _SLV, unroll=8)
    def zs_b(j):
        zslice[pl.ds(j * _L, _L)] = zero16

    pltpu.sync_copy(zslice, sp_acc.at[pl.ds(n0, _SL)])
    plsc.subcore_barrier()

    def _zero_pbuf():
        @plsc.parallel_loop(0, _NP // _L, unroll=16)
        def zb(j):
            pbuf[pl.ds(j * _L, _L)] = zero16

    def _reduce_slice(extra):
        pltpu.sync_copy(pbuf, sp_acc.at[idv], add=True)
        plsc.subcore_barrier()
        pltpu.sync_copy(sp_acc.at[pl.ds(n0, _SL)], accv)
        pltpu.sync_copy(zslice, sp_acc.at[pl.ds(n0, _SL)])

        @plsc.parallel_loop(0, _SLV, unroll=2)
        def rb(j):
            acc = accv[pl.ds(j * _L, _L)]
            extra(j, acc)

    def _publish_vslice():
        pltpu.sync_copy(vslice, sp_v.at[pl.ds(n0, _SL)])
        plsc.subcore_barrier()

    _zero_pbuf()

    @plsc.parallel_loop(0, _EPV, unroll=5)
    def deg_b(j):
        di = dstv[pl.ds(j * _L, _L)]
        plsc.addupdate_scatter(pbuf, [di], one16)

    @plsc.parallel_loop(0, _EPV, unroll=5)
    def pack_b(j):
        sl = pl.ds(j * _L, _L)
        pkv[sl] = jnp.bitwise_or(srcv[sl], jnp.left_shift(dstv[sl], 16))

    def deg_fin(j, acc):
        d = acc + 1.0
        bits = plsc.bitcast(d, jnp.int32)
        y = plsc.bitcast(0x5F3759DF - lax.shift_right_logical(bits, 1),
                         jnp.float32)
        for _ in range(3):
            y = y * (1.5 - 0.5 * d * y * y)
        disv[pl.ds(j * _L, _L)] = y
    _reduce_slice(deg_fin)

    pltpu.sync_copy(v0_h.at[pl.ds(n0, _SL)], vslice)

    @plsc.parallel_loop(0, _SLV, unroll=4)
    def v0s_b(j):
        sl = pl.ds(j * _L, _L)
        vslice[sl] = disv[sl] * vslice[sl]

    _publish_vslice()

    @plsc.parallel_loop(0, _SLV, unroll=4)
    def init_b(j):
        sl = pl.ds(j * _L, _L)
        sumh[sl] = zero16
        stepsv[sl] = one16
        contv[sl] = one16
        kv[sl] = zero16
        pv[sl] = zero16

    for k in range(1, _NI + 1):
        bc = pltpu.async_copy(sp_v, vbuf, dmasem)
        _zero_pbuf()

        @plsc.parallel_loop(0, _SLV, unroll=4)
        def selfb(j):
            sl = pl.ds(j * _L, _L)
            pbuf[pl.ds(n0 + j * _L, _L)] = vslice[sl]
        bc.wait()

        @plsc.parallel_loop(0, _EPV, unroll=5)
        def edge_b(j):
            pk = pkv[pl.ds(j * _L, _L)]
            si = jnp.bitwise_and(pk, 0xFFFF)
            di = lax.shift_right_logical(pk, 16)
            msg = plsc.load_gather(vbuf, [si])
            plsc.addupdate_scatter(pbuf, [di], msg)

        def halt_fin(j, acc, k=k):
            sl = pl.ds(j * _L, _L)
            dd = disv[sl]
            vv = dd * acc
            hh = 1.0 / (1.0 + jnp.exp(-(vv + bv[...])))
            cont = contv[sl]
            sh = sumh[sl]
            st = stepsv[sl]
            prob = jnp.where((sh + hh) < 0.99, cont, zero16)
            st = st + prob
            sh = sh + prob * hh
            cond = jnp.where(st < float(_NI), prob, zero16)
            p = jnp.where(cond > 0.0, sh, 1.0 - sh)
            kv[sl] = jnp.where(cont > 0.0, jnp.full((_L,), float(k)), kv[sl])
            pv[sl] = jnp.where(cont > 0.0, p, pv[sl])
            contv[sl] = prob
            sumh[sl] = sh
            stepsv[sl] = st
            vslice[sl] = dd * vv
        _reduce_slice(halt_fin)
        if k < _NI:
            _publish_vslice()

    @plsc.parallel_loop(0, _SLV, unroll=2)
    def out_b(j):
        sl = pl.ds(j * _L, _L)
        remv[sl] = 1.0 - sumh[sl]
        nid = (n0 + j * _L) + lax.iota(jnp.int32, _L)
        valid = nid < _N
        kk = kv[sl]
        pp = pv[sl]
        for k in range(_NI + 1):
            ck = (jnp.where(kk == float(k), pp, zero16)
                  + jnp.where(kk == float(k + 1), 1.0 - pp, zero16))
            cbuf[pl.ds(k * _SL + j * _L, _L)] = jnp.where(valid, ck, zero16)
    pltpu.sync_copy(stepsv, steps_h.at[pl.ds(n0, _SL)])
    pltpu.sync_copy(remv, rem_h.at[pl.ds(n0, _SL)])

    @plsc.parallel_loop(0, _SLV, unroll=4)
    def s_init(j):
        sl = pl.ds(j * _L, _L)
        vslice[sl] = disv[sl] * cbuf[pl.ds(_NI * _SL + j * _L, _L)]
    _publish_vslice()

    for k in range(_NI - 1, -1, -1):
        bc = pltpu.async_copy(sp_v, vbuf, dmasem)
        _zero_pbuf()

        @plsc.parallel_loop(0, _SLV, unroll=4)
        def selfb2(j):
            sl = pl.ds(j * _L, _L)
            pbuf[pl.ds(n0 + j * _L, _L)] = vslice[sl]
        bc.wait()

        @plsc.parallel_loop(0, _EPV, unroll=5)
        def edge_bt(j):
            pk = pkv[pl.ds(j * _L, _L)]
            si = jnp.bitwise_and(pk, 0xFFFF)
            di = lax.shift_right_logical(pk, 16)
            msg = plsc.load_gather(vbuf, [di])
            plsc.addupdate_scatter(pbuf, [si], msg)

        def horner_fin(j, acc, k=k):
            sl = pl.ds(j * _L, _L)
            sraw = disv[sl] * acc + cbuf[pl.ds(k * _SL + j * _L, _L)]
            if k > 0:
                vslice[sl] = disv[sl] * sraw
            else:
                vslice[sl] = sraw
        _reduce_slice(horner_fin)
        if k > 0:
            _publish_vslice()

    pltpu.sync_copy(vslice, s_h.at[pl.ds(n0, _SL)])


def _sc_propagate(src, dst, v0p, bvec):
    mesh = plsc.VectorSubcoreMesh(core_axis_name="c", subcore_axis_name="s",
                                  num_cores=1, num_subcores=_NT)
    kern = functools.partial(
        pl.kernel,
        mesh=mesh,
        out_type=(jax.ShapeDtypeStruct((_NP,), jnp.float32),
                  jax.ShapeDtypeStruct((_NP,), jnp.float32),
                  jax.ShapeDtypeStruct((_NP,), jnp.float32)),
        compiler_params=pltpu.CompilerParams(needs_layout_passes=False),
        scratch_types=[
            pltpu.VMEM((_EP,), jnp.int32),
            pltpu.VMEM((_EP,), jnp.int32),
            pltpu.VMEM((_EP,), jnp.int32),
            pltpu.VMEM((_NP,), jnp.float32),
            pltpu.VMEM((_NP,), jnp.float32),
            pltpu.VMEM((_NP,), jnp.int32),
            pltpu.VMEM((_SL,), jnp.float32),
            pltpu.VMEM((_SL,), jnp.float32),
            pltpu.VMEM((_SL,), jnp.float32),
            pltpu.VMEM((_SL,), jnp.float32),
            pltpu.VMEM((_SL,), jnp.float32),
            pltpu.VMEM((_SL,), jnp.float32),
            pltpu.VMEM((_SL,), jnp.float32),
            pltpu.VMEM((_SL,), jnp.float32),
            pltpu.VMEM((_SL,), jnp.float32),
            pltpu.VMEM((_SL,), jnp.float32),
            pltpu.VMEM(((_NI + 1) * _SL,), jnp.float32),
            pltpu.VMEM((_L,), jnp.float32),
            pltpu.VMEM_SHARED((_NP,), jnp.float32),
            pltpu.VMEM_SHARED((_NP,), jnp.float32),
            pltpu.SemaphoreType.DMA,
        ])(_sc_body)
    return kern(src, dst, v0p, bvec)


def kernel(h, e, snorm_n, snorm_e, W_emb, b_emb, W_l0, b_l0, W_halt, b_halt,
           Wr0, br0, Wr1, br1, Wr2, br2, edge_index):
    del e, snorm_n, snorm_e
    src = edge_index[0]
    dst = edge_index[1]

    v0 = _compute_v0(h, W_emb, b_emb, W_l0, b_l0, W_halt)
    v0p = jnp.concatenate([v0[:, 0], jnp.zeros((_NP - _N,), jnp.float32)])
    bvec = jnp.broadcast_to(b_halt, (_L,)).astype(jnp.float32)

    steps_p, rem_p, s_p = _sc_propagate(src, dst, v0p, bvec)

    scores = _readout(s_p[:_N, None], h, W_emb, b_emb, W_l0, b_l0,
                      Wr0, br0, Wr1, br1, Wr2, br2)
    return scores, steps_p[:_N], rem_p[:_N]


# speedup vs baseline: 1.2776x; 1.2776x over previous
import functools

import jax
import jax.numpy as jnp
from jax import lax
from jax.experimental import pallas as pl
from jax.experimental.pallas import tpu as pltpu
from jax.experimental.pallas import tpu_sc as plsc

_N = 10000
_E = 160000
_NI = 10
_HID = 256
_L = 16
_NT = 16
_NP = 10240
_SL = _NP // _NT
_SLV = _SL // _L
_EP = _E // _NT
_EPV = _EP // _L


def _v0_body(h_ref, we_ref, be_ref, wl_ref, bl_ref, wh_ref, bh_ref,
             out_ref, bout_ref):
    u = jnp.dot(wl_ref[...], wh_ref[...], preferred_element_type=jnp.float32)
    wv = jnp.dot(we_ref[...], u, preferred_element_type=jnp.float32)
    cv = (jnp.dot(be_ref[...], u, preferred_element_type=jnp.float32)
          + jnp.dot(bl_ref[...], wh_ref[...], preferred_element_type=jnp.float32))
    out_ref[...] = jnp.dot(h_ref[...], wv, preferred_element_type=jnp.float32) + cv
    bout_ref[...] = jnp.broadcast_to(bh_ref[...], (1, _L))


def _compute_v0(h, W_emb, b_emb, W_l0, b_l0, W_halt, b_halt):
    nb = 10
    rb = _N // nb
    full = lambda shape: pl.BlockSpec(shape, lambda i: (0, 0))
    return pl.pallas_call(
        _v0_body,
        grid=(nb,),
        in_specs=[
            pl.BlockSpec((rb, _HID), lambda i: (i, 0)),
            full((_HID, _HID)),
            full((1, _HID)),
            full((_HID, _HID)),
            full((1, _HID)),
            full((_HID, 1)),
            full((1, 1)),
        ],
        out_specs=[pl.BlockSpec((rb, 1), lambda i: (i, 0)),
                   full((1, _L))],
        out_shape=[jax.ShapeDtypeStruct((_NP, 1), jnp.float32),
                   jax.ShapeDtypeStruct((1, _L), jnp.float32)],
    )(h, W_emb, b_emb[None, :], W_l0, b_l0[None, :], W_halt, b_halt[None, :])


def _readout_body(s_ref, h_ref, we_ref, be_ref, wl_ref, bl_ref,
                  wr0_ref, br0_ref, wr1_ref, br1_ref, wr2_ref, br2_ref,
                  out_ref, t_acc, ss_acc):
    g = pl.program_id(0)

    @pl.when(g == 0)
    def _():
        t_acc[...] = jnp.zeros_like(t_acc)
        ss_acc[...] = jnp.zeros_like(ss_acc)

    t_acc[...] += lax.dot_general(
        s_ref[...], h_ref[...], (((0,), (0,)), ((), ())),
        preferred_element_type=jnp.float32)
    ss_acc[...] += jnp.sum(s_ref[...]).reshape(1, 1)

    @pl.when(g == pl.num_programs(0) - 1)
    def _():
        t = t_acc[...]
        ss = ss_acc[...]
        tw = jnp.dot(jnp.dot(t, we_ref[...], preferred_element_type=jnp.float32),
                     wl_ref[...], preferred_element_type=jnp.float32)
        bb = (jnp.dot(be_ref[...], wl_ref[...], preferred_element_type=jnp.float32)
              + bl_ref[...])
        hg = (tw + ss * bb) * (1.0 / _N)
        y = jnp.maximum(
            jnp.dot(hg, wr0_ref[...], preferred_element_type=jnp.float32)
            + br0_ref[...], 0.0)
        y = jnp.maximum(
            jnp.dot(y, wr1_ref[...], preferred_element_type=jnp.float32)
            + br1_ref[...], 0.0)
        out_ref[...] = (jnp.dot(y, wr2_ref[...], preferred_element_type=jnp.float32)
                        + br2_ref[...])


def _readout(s, h, W_emb, b_emb, W_l0, b_l0, Wr0, br0, Wr1, br1, Wr2, br2):
    nb = 10
    rb = _N // nb
    full = lambda shape: pl.BlockSpec(shape, lambda i: (0, 0))
    return pl.pallas_call(
        _readout_body,
        grid=(nb,),
        in_specs=[
            pl.BlockSpec((rb, 1), lambda i: (i, 0)),
            pl.BlockSpec((rb, _HID), lambda i: (i, 0)),
            full((_HID, _HID)),
            full((1, _HID)),
            full((_HID, _HID)),
            full((1, _HID)),
            full((_HID, _HID // 2)),
            full((1, _HID // 2)),
            full((_HID // 2, _HID // 4)),
            full((1, _HID // 4)),
            full((_HID // 4, 10)),
            full((1, 10)),
        ],
        out_specs=full((1, 10)),
        out_shape=jax.ShapeDtypeStruct((1, 10), jnp.float32),
        scratch_shapes=[
            pltpu.VMEM((1, _HID), jnp.float32),
            pltpu.VMEM((1, 1), jnp.float32),
        ],
    )(s, h, W_emb, b_emb[None, :], W_l0, b_l0[None, :],
      Wr0, br0[None, :], Wr1, br1[None, :], Wr2, br2[None, :])


def _sc_body(src_h, dst_h, v0_h, b_h,
             steps_h, rem_h, s_h,
             srcv, dstv, pkv, vbuf, pbuf, tmp,
             disv, sumh, stepsv, contv, kv, pv, vslice, remv, cbuf, bv,
             sp_part, sp_v, dmasem):
    tid = lax.axis_index("s")
    e0 = tid * _EP
    n0 = tid * _SL

    pltpu.sync_copy(src_h.at[pl.ds(e0, _EP)], srcv)
    pltpu.sync_copy(dst_h.at[pl.ds(e0, _EP)], dstv)
    pltpu.sync_copy(b_h, bv)

    zero16 = jnp.zeros((_L,), jnp.float32)
    one16 = jnp.ones((_L,), jnp.float32)

    def _zero_pbuf():
        @plsc.parallel_loop(0, _NP // _L, unroll=16)
        def zb(j):
            pbuf[pl.ds(j * _L, _L)] = zero16

    def _reduce_slice(extra):
        pltpu.sync_copy(pbuf, sp_part.at[tid])
        plsc.subcore_barrier()
        pltpu.sync_copy(sp_part.at[:, pl.ds(n0, _SL)], tmp)

        @plsc.parallel_loop(0, _SLV, unroll=2)
        def rb(j):
            acc = tmp[0, pl.ds(j * _L, _L)]
            for t in range(1, _NT):
                acc = acc + tmp[t, pl.ds(j * _L, _L)]
            extra(j, acc)

    def _publish_vslice():
        pltpu.sync_copy(vslice, sp_v.at[pl.ds(n0, _SL)])
        plsc.subcore_barrier()

    _zero_pbuf()

    @plsc.parallel_loop(0, _EPV, unroll=5)
    def deg_b(j):
        di = dstv[pl.ds(j * _L, _L)]
        plsc.addupdate_scatter(pbuf, [di], one16)

    @plsc.parallel_loop(0, _EPV, unroll=5)
    def pack_b(j):
        sl = pl.ds(j * _L, _L)
        pkv[sl] = jnp.bitwise_or(srcv[sl], jnp.left_shift(dstv[sl], 16))

    def deg_fin(j, acc):
        d = acc + 1.0
        bits = plsc.bitcast(d, jnp.int32)
        y = plsc.bitcast(0x5F3759DF - lax.shift_right_logical(bits, 1),
                         jnp.float32)
        for _ in range(3):
            y = y * (1.5 - 0.5 * d * y * y)
        disv[pl.ds(j * _L, _L)] = y
    _reduce_slice(deg_fin)

    pltpu.sync_copy(v0_h.at[pl.ds(n0, _SL)], vslice)

    @plsc.parallel_loop(0, _SLV, unroll=4)
    def v0s_b(j):
        sl = pl.ds(j * _L, _L)
        vslice[sl] = disv[sl] * vslice[sl]

    _publish_vslice()

    @plsc.parallel_loop(0, _SLV, unroll=4)
    def init_b(j):
        sl = pl.ds(j * _L, _L)
        sumh[sl] = zero16
        stepsv[sl] = one16
        contv[sl] = one16
        kv[sl] = zero16
        pv[sl] = zero16

    for k in range(1, _NI + 1):
        bc = pltpu.async_copy(sp_v, vbuf, dmasem)
        _zero_pbuf()

        @plsc.parallel_loop(0, _SLV, unroll=4)
        def selfb(j):
            sl = pl.ds(j * _L, _L)
            pbuf[pl.ds(n0 + j * _L, _L)] = vslice[sl]
        bc.wait()

        @plsc.parallel_loop(0, _EPV, unroll=5)
        def edge_b(j):
            pk = pkv[pl.ds(j * _L, _L)]
            si = jnp.bitwise_and(pk, 0xFFFF)
            di = lax.shift_right_logical(pk, 16)
            msg = plsc.load_gather(vbuf, [si])
            plsc.addupdate_scatter(pbuf, [di], msg)

        def halt_fin(j, acc, k=k):
            sl = pl.ds(j * _L, _L)
            dd = disv[sl]
            vv = dd * acc
            hh = 1.0 / (1.0 + jnp.exp(-(vv + bv[...])))
            cont = contv[sl]
            sh = sumh[sl]
            st = stepsv[sl]
            prob = jnp.where((sh + hh) < 0.99, cont, zero16)
            st = st + prob
            sh = sh + prob * hh
            cond = jnp.where(st < float(_NI), prob, zero16)
            p = jnp.where(cond > 0.0, sh, 1.0 - sh)
            kv[sl] = jnp.where(cont > 0.0, jnp.full((_L,), float(k)), kv[sl])
            pv[sl] = jnp.where(cont > 0.0, p, pv[sl])
            contv[sl] = prob
            sumh[sl] = sh
            stepsv[sl] = st
            vslice[sl] = dd * vv
        _reduce_slice(halt_fin)
        if k < _NI:
            _publish_vslice()

    @plsc.parallel_loop(0, _SLV, unroll=2)
    def out_b(j):
        sl = pl.ds(j * _L, _L)
        remv[sl] = 1.0 - sumh[sl]
        nid = (n0 + j * _L) + lax.iota(jnp.int32, _L)
        valid = nid < _N
        kk = kv[sl]
        pp = pv[sl]
        for k in range(_NI + 1):
            ck = (jnp.where(kk == float(k), pp, zero16)
                  + jnp.where(kk == float(k + 1), 1.0 - pp, zero16))
            cbuf[pl.ds(k * _SL + j * _L, _L)] = jnp.where(valid, ck, zero16)
    _TL = _N - (_NT - 1) * _SL

    @pl.when(tid < _NT - 1)
    def _():
        pltpu.sync_copy(stepsv, steps_h.at[pl.ds(n0, _SL)])
        pltpu.sync_copy(remv, rem_h.at[pl.ds(n0, _SL)])

    @pl.when(tid == _NT - 1)
    def _():
        pltpu.sync_copy(stepsv.at[pl.ds(0, _TL)],
                        steps_h.at[pl.ds((_NT - 1) * _SL, _TL)])
        pltpu.sync_copy(remv.at[pl.ds(0, _TL)],
                        rem_h.at[pl.ds((_NT - 1) * _SL, _TL)])

    @plsc.parallel_loop(0, _SLV, unroll=4)
    def s_init(j):
        sl = pl.ds(j * _L, _L)
        vslice[sl] = disv[sl] * cbuf[pl.ds(_NI * _SL + j * _L, _L)]
    _publish_vslice()

    for k in range(_NI - 1, -1, -1):
        bc = pltpu.async_copy(sp_v, vbuf, dmasem)
        _zero_pbuf()

        @plsc.parallel_loop(0, _SLV, unroll=4)
        def selfb2(j):
            sl = pl.ds(j * _L, _L)
            pbuf[pl.ds(n0 + j * _L, _L)] = vslice[sl]
        bc.wait()

        @plsc.parallel_loop(0, _EPV, unroll=5)
        def edge_bt(j):
            pk = pkv[pl.ds(j * _L, _L)]
            si = jnp.bitwise_and(pk, 0xFFFF)
            di = lax.shift_right_logical(pk, 16)
            msg = plsc.load_gather(vbuf, [di])
            plsc.addupdate_scatter(pbuf, [si], msg)

        def horner_fin(j, acc, k=k):
            sl = pl.ds(j * _L, _L)
            sraw = disv[sl] * acc + cbuf[pl.ds(k * _SL + j * _L, _L)]
            if k > 0:
                vslice[sl] = disv[sl] * sraw
            else:
                vslice[sl] = sraw
        _reduce_slice(horner_fin)
        if k > 0:
            _publish_vslice()

    @pl.when(tid < _NT - 1)
    def _():
        pltpu.sync_copy(vslice, s_h.at[pl.ds(n0, _SL)])

    @pl.when(tid == _NT - 1)
    def _():
        pltpu.sync_copy(vslice.at[pl.ds(0, _TL)],
                        s_h.at[pl.ds((_NT - 1) * _SL, _TL)])


def _sc_propagate(src, dst, v0p, bvec):
    mesh = plsc.VectorSubcoreMesh(core_axis_name="c", subcore_axis_name="s",
                                  num_cores=1, num_subcores=_NT)
    kern = functools.partial(
        pl.kernel,
        mesh=mesh,
        out_type=(jax.ShapeDtypeStruct((_N,), jnp.float32),
                  jax.ShapeDtypeStruct((_N,), jnp.float32),
                  jax.ShapeDtypeStruct((_N,), jnp.float32)),
        compiler_params=pltpu.CompilerParams(needs_layout_passes=False),
        scratch_types=[
            pltpu.VMEM((_EP,), jnp.int32),
            pltpu.VMEM((_EP,), jnp.int32),
            pltpu.VMEM((_EP,), jnp.int32),
            pltpu.VMEM((_NP,), jnp.float32),
            pltpu.VMEM((_NP,), jnp.float32),
            pltpu.VMEM((_NT, _SL), jnp.float32),
            pltpu.VMEM((_SL,), jnp.float32),
            pltpu.VMEM((_SL,), jnp.float32),
            pltpu.VMEM((_SL,), jnp.float32),
            pltpu.VMEM((_SL,), jnp.float32),
            pltpu.VMEM((_SL,), jnp.float32),
            pltpu.VMEM((_SL,), jnp.float32),
            pltpu.VMEM((_SL,), jnp.float32),
            pltpu.VMEM((_SL,), jnp.float32),
            pltpu.VMEM(((_NI + 1) * _SL,), jnp.float32),
            pltpu.VMEM((_L,), jnp.float32),
            pltpu.VMEM_SHARED((_NT, _NP), jnp.float32),
            pltpu.VMEM_SHARED((_NP,), jnp.float32),
            pltpu.SemaphoreType.DMA,
        ])(_sc_body)
    return kern(src, dst, v0p, bvec)


def kernel(h, e, snorm_n, snorm_e, W_emb, b_emb, W_l0, b_l0, W_halt, b_halt,
           Wr0, br0, Wr1, br1, Wr2, br2, edge_index):
    del e, snorm_n, snorm_e
    src = edge_index[0]
    dst = edge_index[1]

    v0p, bvec = _compute_v0(h, W_emb, b_emb, W_l0, b_l0, W_halt, b_halt)

    steps, rem, s = _sc_propagate(src, dst, jnp.reshape(v0p, (_NP,)),
                                  jnp.reshape(bvec, (_L,)))

    scores = _readout(jnp.reshape(s, (_N, 1)), h, W_emb, b_emb, W_l0, b_l0,
                      Wr0, br0, Wr1, br1, Wr2, br2)
    return scores, steps, rem
